# baseline (speedup 1.0000x reference)
import functools

import jax
import jax.numpy as jnp
from jax import lax
from jax.experimental import pallas as pl
from jax.experimental.pallas import tpu as pltpu
from jax.experimental.pallas import tpu_sc as plsc

N_NODES = 10000
N_REL = 500
R_PAD = 512
H_DIM = 128
N_EDGES = 320000

N_SC = 2
N_TILES = 16
N_CHUNKS = 4
CHUNK_NODES = N_NODES // N_CHUNKS
CHUNK_WORDS = CHUNK_NODES * R_PAD
TABLE_WORDS = CHUNK_WORDS + 8
EDGES_PER_TILE = N_EDGES // N_TILES
BATCH = 128
N_BATCH = (EDGES_PER_TILE + BATCH - 1) // BATCH
STAGE = N_BATCH * BATCH
TILE_WORDS = CHUNK_WORDS // N_TILES
PIECE = 3200
N_PIECE = TILE_WORDS // PIECE
RSTAGE = 2000


def _hist_body(dst_hbm, rel_hbm, out_hbm, key_v, idx_v, rel_s, ones_v,
               znc_v, table_sh, sem_a, sem_s):
    c = lax.axis_index("c")
    s = lax.axis_index("s")

    base = s * EDGES_PER_TILE
    pltpu.sync_copy(dst_hbm.at[pl.ds(base, EDGES_PER_TILE)],
                    key_v.at[pl.ds(0, EDGES_PER_TILE)])
    for p in range(EDGES_PER_TILE // RSTAGE):
        pltpu.sync_copy(rel_hbm.at[pl.ds(base + p * RSTAGE, RSTAGE)], rel_s)

        def _fold(i, carry, p=p):
            off = p * RSTAGE + i * 16
            key_v[pl.ds(off, 16)] = (
                key_v[pl.ds(off, 16)] * R_PAD + rel_s[pl.ds(i * 16, 16)])
            return carry

        lax.fori_loop(0, RSTAGE // 16, _fold, 0)
    neg1 = jnp.full((16,), -1, jnp.int32)
    for k in range(EDGES_PER_TILE, STAGE, 16):
        key_v[pl.ds(k, 16)] = neg1

    one16 = jnp.ones((16,), jnp.float32)
    for k in range(0, BATCH, 16):
        ones_v[pl.ds(k, 16)] = one16

    zero16 = jnp.zeros((16,), jnp.float32)

    def _zfill(i, carry):
        znc_v[pl.ds(i * 16, 16)] = zero16
        return carry

    for cc in range(N_CHUNKS // N_SC):
        chunk = c * (N_CHUNKS // N_SC) + cc
        c0w = chunk * CHUNK_WORDS

        lax.fori_loop(0, (2 * PIECE) // 16, _zfill, 0)
        zdmas = []
        for z in range(N_PIECE):
            if z >= 4:
                zdmas[z - 4].wait()
            zdmas.append(pltpu.async_copy(
                znc_v.at[pl.ds((z % 2) * PIECE, PIECE)],
                table_sh.at[pl.ds(s * TILE_WORDS + z * PIECE, PIECE)], sem_a))

        for d in zdmas[-4:]:
            d.wait()
        plsc.subcore_barrier()

        def _batch(j, carry):
            for k in range(0, BATCH, 16):
                t = key_v[pl.ds(j * BATCH + k, 16)] - c0w
                ok = (t >= 0) & (t < CHUNK_WORDS)
                idx_v[0, pl.ds(k, 16)] = jnp.where(ok, t, CHUNK_WORDS)
            pltpu.sync_copy(ones_v, table_sh.at[idx_v.at[0]], add=True)
            return carry

        lax.fori_loop(0, N_BATCH, _batch, 0)
        plsc.subcore_barrier()

        out_base = chunk * CHUNK_WORDS + s * TILE_WORDS
        outs = [None, None]
        for z in range(N_PIECE):
            b = z % 2
            if outs[b] is not None:
                outs[b].wait()
            pltpu.async_copy(
                table_sh.at[pl.ds(s * TILE_WORDS + z * PIECE, PIECE)],
                znc_v.at[pl.ds(b * PIECE, PIECE)], sem_a).wait()
            outs[b] = pltpu.async_copy(
                znc_v.at[pl.ds(b * PIECE, PIECE)],
                out_hbm.at[pl.ds(out_base + z * PIECE, PIECE)], sem_s)
        for o in outs:
            if o is not None:
                o.wait()
        plsc.subcore_barrier()


@functools.cache
def _make_hist():
  return pl.kernel(
    _hist_body,
    out_type=jax.ShapeDtypeStruct((N_NODES * R_PAD,), jnp.float32),
    mesh=plsc.VectorSubcoreMesh(core_axis_name="c", subcore_axis_name="s"),
    scratch_types=[
        pltpu.VMEM((STAGE,), jnp.int32),
        pltpu.VMEM((1, BATCH), jnp.int32),
        pltpu.VMEM((RSTAGE,), jnp.int32),
        pltpu.VMEM((BATCH,), jnp.float32),
        pltpu.VMEM((2 * PIECE,), jnp.float32),
        pltpu.VMEM_SHARED((TABLE_WORDS,), jnp.float32),
        pltpu.SemaphoreType.DMA,
        pltpu.SemaphoreType.DMA,
    ],
  )


def _dense_body(ent_ref, cnt_ref, relp_ref, nw_ref, out_ref):
    ent = ent_ref[...]
    cnt = cnt_ref[...]
    relp = relp_ref[...]
    logits = lax.dot_general(ent, relp, (((1,), (1,)), ((), ())),
                             preferred_element_type=jnp.float32,
                             precision=lax.Precision.HIGHEST)
    mask = cnt > 0.0
    m = jnp.max(jnp.where(mask, logits, -jnp.inf), axis=1, keepdims=True)
    ex = jnp.exp(jnp.where(mask, logits - m, -30.0))
    p = cnt * ex
    ssum = jnp.sum(p, axis=1, keepdims=True)
    w = jnp.where(ssum > 0.0, p / ssum, 0.0)
    neigh = jnp.dot(w, relp, preferred_element_type=jnp.float32,
                    precision=lax.Precision.HIGHEST)
    out_ref[...] = jnp.tanh(jnp.dot(neigh, nw_ref[...],
                                    preferred_element_type=jnp.float32,
                                    precision=lax.Precision.HIGHEST))


BLK = 1000

_dense = pl.pallas_call(
    _dense_body,
    grid=(N_NODES // BLK,),
    in_specs=[
        pl.BlockSpec((BLK, H_DIM), lambda i: (i, 0)),
        pl.BlockSpec((BLK, R_PAD), lambda i: (i, 0)),
        pl.BlockSpec((R_PAD, H_DIM), lambda i: (0, 0)),
        pl.BlockSpec((H_DIM, H_DIM), lambda i: (0, 0)),
    ],
    out_specs=pl.BlockSpec((BLK, H_DIM), lambda i: (i, 0)),
    out_shape=jax.ShapeDtypeStruct((N_NODES, H_DIM), jnp.float32),
)


def kernel(ent_emb, rel_emb, neigh_w, edge_index, rel_id):
    dst = edge_index[1]
    relp = jnp.zeros((R_PAD, H_DIM), jnp.float32).at[:N_REL].set(rel_emb)
    cnt = _make_hist()(dst.astype(jnp.int32), rel_id.astype(jnp.int32)
                       ).reshape(N_NODES, R_PAD)
    return _dense(ent_emb, cnt, relp, neigh_w)


# speedup vs baseline: 3.3260x; 3.3260x over previous
import functools

import jax
import jax.numpy as jnp
from jax import lax
from jax.experimental import pallas as pl
from jax.experimental.pallas import tpu as pltpu
from jax.experimental.pallas import tpu_sc as plsc

N_NODES = 10000
N_REL = 500
R_PAD = 512
H_DIM = 128
N_EDGES = 320000

N_SC = 2
N_TILES = 16
N_CHUNKS = 4
CHUNK_NODES = N_NODES // N_CHUNKS
CHUNK_WORDS = CHUNK_NODES * R_PAD
TABLE_WORDS = CHUNK_WORDS + 128
EDGES_PER_TILE = N_EDGES // N_TILES
BATCH = 128
N_BATCH = (EDGES_PER_TILE + BATCH - 1) // BATCH
STAGE = N_BATCH * BATCH
TILE_WORDS = CHUNK_WORDS // N_TILES
PIECE = 3200
N_PIECE = TILE_WORDS // PIECE
RSTAGE = 2000


def _hist_body(dst_hbm, rel_hbm, out_hbm, key_v, idx_v, rel_s, ones_v,
               znc_v, table_sh, sem_a, sem_s):
    c = lax.axis_index("c")
    s = lax.axis_index("s")

    base = s * EDGES_PER_TILE
    pltpu.sync_copy(dst_hbm.at[pl.ds(base, EDGES_PER_TILE)],
                    key_v.at[pl.ds(0, EDGES_PER_TILE)])
    for p in range(EDGES_PER_TILE // RSTAGE):
        pltpu.sync_copy(rel_hbm.at[pl.ds(base + p * RSTAGE, RSTAGE)], rel_s)

        def _fold(i, carry, p=p):
            off = p * RSTAGE + i * 16
            key_v[pl.ds(off, 16)] = (
                key_v[pl.ds(off, 16)] * R_PAD + rel_s[pl.ds(i * 16, 16)])
            return carry

        lax.fori_loop(0, RSTAGE // 16, _fold, 0)
    neg1 = jnp.full((16,), -1, jnp.int32)
    for k in range(EDGES_PER_TILE, STAGE, 16):
        key_v[pl.ds(k, 16)] = neg1

    one16 = jnp.ones((16,), jnp.float32)
    for k in range(0, BATCH, 16):
        ones_v[pl.ds(k, 16)] = one16

    zero16 = jnp.zeros((16,), jnp.float32)

    def _zfill(i, carry):
        znc_v[pl.ds(i * 16, 16)] = zero16
        return carry

    for cc in range(N_CHUNKS // N_SC):
        chunk = c * (N_CHUNKS // N_SC) + cc
        c0w = chunk * CHUNK_WORDS

        lax.fori_loop(0, (2 * PIECE) // 16, _zfill, 0)
        zdmas = []
        for z in range(N_PIECE):
            if z >= 4:
                zdmas[z - 4].wait()
            zdmas.append(pltpu.async_copy(
                znc_v.at[pl.ds((z % 2) * PIECE, PIECE)],
                table_sh.at[pl.ds(s * TILE_WORDS + z * PIECE, PIECE)], sem_a))

        for d in zdmas[-4:]:
            d.wait()
        plsc.subcore_barrier()

        def _batch(j, carry):
            for k in range(0, BATCH, 16):
                t = key_v[pl.ds(j * BATCH + k, 16)] - c0w
                ok = (t >= 0) & (t < CHUNK_WORDS)
                dummy = lax.iota(jnp.int32, 16) + (CHUNK_WORDS + k)
                idx_v[0, pl.ds(k, 16)] = jnp.where(ok, t, dummy)
            pltpu.sync_copy(ones_v, table_sh.at[idx_v.at[0]], add=True)
            return carry

        lax.fori_loop(0, N_BATCH, _batch, 0)
        plsc.subcore_barrier()

        out_base = chunk * CHUNK_WORDS + s * TILE_WORDS
        outs = [None, None]
        for z in range(N_PIECE):
            b = z % 2
            if outs[b] is not None:
                outs[b].wait()
            pltpu.async_copy(
                table_sh.at[pl.ds(s * TILE_WORDS + z * PIECE, PIECE)],
                znc_v.at[pl.ds(b * PIECE, PIECE)], sem_a).wait()
            outs[b] = pltpu.async_copy(
                znc_v.at[pl.ds(b * PIECE, PIECE)],
                out_hbm.at[pl.ds(out_base + z * PIECE, PIECE)], sem_s)
        for o in outs:
            if o is not None:
                o.wait()
        plsc.subcore_barrier()


@functools.cache
def _make_hist():
  return pl.kernel(
    _hist_body,
    out_type=jax.ShapeDtypeStruct((N_NODES * R_PAD,), jnp.float32),
    mesh=plsc.VectorSubcoreMesh(core_axis_name="c", subcore_axis_name="s"),
    scratch_types=[
        pltpu.VMEM((STAGE,), jnp.int32),
        pltpu.VMEM((1, BATCH), jnp.int32),
        pltpu.VMEM((RSTAGE,), jnp.int32),
        pltpu.VMEM((BATCH,), jnp.float32),
        pltpu.VMEM((2 * PIECE,), jnp.float32),
        pltpu.VMEM_SHARED((TABLE_WORDS,), jnp.float32),
        pltpu.SemaphoreType.DMA,
        pltpu.SemaphoreType.DMA,
    ],
  )


def _dense_body(ent_ref, cnt_ref, relp_ref, nw_ref, out_ref):
    ent = ent_ref[...]
    cnt = cnt_ref[...]
    relp = relp_ref[...]
    logits = lax.dot_general(ent, relp, (((1,), (1,)), ((), ())),
                             preferred_element_type=jnp.float32,
                             precision=lax.Precision.HIGHEST)
    mask = cnt > 0.0
    m = jnp.max(jnp.where(mask, logits, -jnp.inf), axis=1, keepdims=True)
    ex = jnp.exp(jnp.where(mask, logits - m, -30.0))
    p = cnt * ex
    ssum = jnp.sum(p, axis=1, keepdims=True)
    w = jnp.where(ssum > 0.0, p / ssum, 0.0)
    neigh = jnp.dot(w, relp, preferred_element_type=jnp.float32,
                    precision=lax.Precision.HIGHEST)
    out_ref[...] = jnp.tanh(jnp.dot(neigh, nw_ref[...],
                                    preferred_element_type=jnp.float32,
                                    precision=lax.Precision.HIGHEST))


BLK = 1000

_dense = pl.pallas_call(
    _dense_body,
    grid=(N_NODES // BLK,),
    in_specs=[
        pl.BlockSpec((BLK, H_DIM), lambda i: (i, 0)),
        pl.BlockSpec((BLK, R_PAD), lambda i: (i, 0)),
        pl.BlockSpec((R_PAD, H_DIM), lambda i: (0, 0)),
        pl.BlockSpec((H_DIM, H_DIM), lambda i: (0, 0)),
    ],
    out_specs=pl.BlockSpec((BLK, H_DIM), lambda i: (i, 0)),
    out_shape=jax.ShapeDtypeStruct((N_NODES, H_DIM), jnp.float32),
)


def kernel(ent_emb, rel_emb, neigh_w, edge_index, rel_id):
    dst = edge_index[1]
    relp = jnp.zeros((R_PAD, H_DIM), jnp.float32).at[:N_REL].set(rel_emb)
    cnt = _make_hist()(dst.astype(jnp.int32), rel_id.astype(jnp.int32)
                       ).reshape(N_NODES, R_PAD)
    return _dense(ent_emb, cnt, relp, neigh_w)


# speedup vs baseline: 4.2554x; 1.2794x over previous
import functools

import jax
import jax.numpy as jnp
from jax import lax
from jax.experimental import pallas as pl
from jax.experimental.pallas import tpu as pltpu
from jax.experimental.pallas import tpu_sc as plsc

N_NODES = 10000
N_REL = 500
R_PAD = 512
H_DIM = 128
N_EDGES = 320000

N_SC = 2
N_TILES = 16
N_CHUNKS = 4
CHUNK_NODES = N_NODES // N_CHUNKS
CHUNK_WORDS = CHUNK_NODES * R_PAD
TABLE_WORDS = CHUNK_WORDS + 128
EDGES_PER_TILE = N_EDGES // N_TILES
BATCH = 128
N_BATCH = (EDGES_PER_TILE + BATCH - 1) // BATCH
STAGE = N_BATCH * BATCH
TILE_WORDS = CHUNK_WORDS // N_TILES
PIECE = 3200
N_PIECE = TILE_WORDS // PIECE
RSTAGE = 2000


def _hist_body(dst_hbm, rel_hbm, out_hbm, key_v, idx_v, rel_s, ones_v,
               znc_v, table_sh, sem_a, sem_s):
    c = lax.axis_index("c")
    s = lax.axis_index("s")

    base = s * EDGES_PER_TILE
    pltpu.sync_copy(dst_hbm.at[pl.ds(base, EDGES_PER_TILE)],
                    key_v.at[pl.ds(0, EDGES_PER_TILE)])
    for p in range(EDGES_PER_TILE // RSTAGE):
        pltpu.sync_copy(rel_hbm.at[pl.ds(base + p * RSTAGE, RSTAGE)], rel_s)

        def _fold(i, carry, p=p):
            off = p * RSTAGE + i * 16
            key_v[pl.ds(off, 16)] = (
                key_v[pl.ds(off, 16)] * R_PAD + rel_s[pl.ds(i * 16, 16)])
            return carry

        lax.fori_loop(0, RSTAGE // 16, _fold, 0)
    neg1 = jnp.full((16,), -1, jnp.int32)
    for k in range(EDGES_PER_TILE, STAGE, 16):
        key_v[pl.ds(k, 16)] = neg1

    one16 = jnp.ones((16,), jnp.float32)
    for k in range(0, BATCH, 16):
        ones_v[pl.ds(k, 16)] = one16

    zero16 = jnp.zeros((16,), jnp.float32)

    def _zfill(i, carry):
        znc_v[pl.ds(i * 16, 16)] = zero16
        return carry

    for cc in range(N_CHUNKS // N_SC):
        chunk = c * (N_CHUNKS // N_SC) + cc
        c0w = chunk * CHUNK_WORDS

        lax.fori_loop(0, (2 * PIECE) // 16, _zfill, 0)
        zdmas = []
        for z in range(N_PIECE):
            if z >= 4:
                zdmas[z - 4].wait()
            zdmas.append(pltpu.async_copy(
                znc_v.at[pl.ds((z % 2) * PIECE, PIECE)],
                table_sh.at[pl.ds(s * TILE_WORDS + z * PIECE, PIECE)], sem_a))

        for d in zdmas[-4:]:
            d.wait()
        plsc.subcore_barrier()

        def _batch(j, carry):
            for k in range(0, BATCH, 16):
                t = key_v[pl.ds(j * BATCH + k, 16)] - c0w
                ok = (t >= 0) & (t < CHUNK_WORDS)
                dummy = lax.iota(jnp.int32, 16) + (CHUNK_WORDS + k)
                idx_v[0, pl.ds(k, 16)] = jnp.where(ok, t, dummy)
            pltpu.sync_copy(ones_v, table_sh.at[idx_v.at[0]], add=True)
            return carry

        lax.fori_loop(0, N_BATCH, _batch, 0)
        plsc.subcore_barrier()

        out_base = chunk * CHUNK_WORDS + s * TILE_WORDS
        outs = [None, None]
        for z in range(N_PIECE):
            b = z % 2
            if outs[b] is not None:
                outs[b].wait()
            pltpu.async_copy(
                table_sh.at[pl.ds(s * TILE_WORDS + z * PIECE, PIECE)],
                znc_v.at[pl.ds(b * PIECE, PIECE)], sem_a).wait()
            outs[b] = pltpu.async_copy(
                znc_v.at[pl.ds(b * PIECE, PIECE)],
                out_hbm.at[pl.ds(out_base + z * PIECE, PIECE)], sem_s)
        for o in outs:
            if o is not None:
                o.wait()
        plsc.subcore_barrier()


@functools.cache
def _make_hist():
  return pl.kernel(
    _hist_body,
    out_type=jax.ShapeDtypeStruct((N_NODES * R_PAD,), jnp.float32),
    mesh=plsc.VectorSubcoreMesh(core_axis_name="c", subcore_axis_name="s"),
    scratch_types=[
        pltpu.VMEM((STAGE,), jnp.int32),
        pltpu.VMEM((1, BATCH), jnp.int32),
        pltpu.VMEM((RSTAGE,), jnp.int32),
        pltpu.VMEM((BATCH,), jnp.float32),
        pltpu.VMEM((2 * PIECE,), jnp.float32),
        pltpu.VMEM_SHARED((TABLE_WORDS,), jnp.float32),
        pltpu.SemaphoreType.DMA,
        pltpu.SemaphoreType.DMA,
    ],
  )


def _dense_body(ent_ref, cnt_ref, relp_ref, nw_ref, out_ref):
    ent = ent_ref[...]
    cnt = cnt_ref[...]
    relp = jnp.concatenate(
        [relp_ref[...], jnp.zeros((R_PAD - N_REL, H_DIM), jnp.float32)],
        axis=0)
    logits = lax.dot_general(ent, relp, (((1,), (1,)), ((), ())),
                             preferred_element_type=jnp.float32,
                             precision=lax.Precision.HIGHEST)
    mask = cnt > 0.0
    m = jnp.max(jnp.where(mask, logits, -jnp.inf), axis=1, keepdims=True)
    ex = jnp.exp(jnp.where(mask, logits - m, -30.0))
    p = cnt * ex
    ssum = jnp.sum(p, axis=1, keepdims=True)
    w = jnp.where(ssum > 0.0, p / ssum, 0.0)
    neigh = jnp.dot(w, relp, preferred_element_type=jnp.float32)
    out_ref[...] = jnp.tanh(jnp.dot(neigh, nw_ref[...],
                                    preferred_element_type=jnp.float32))


BLK = 1000

_dense = pl.pallas_call(
    _dense_body,
    grid=(N_NODES // BLK,),
    in_specs=[
        pl.BlockSpec((BLK, H_DIM), lambda i: (i, 0)),
        pl.BlockSpec((BLK, R_PAD), lambda i: (i, 0)),
        pl.BlockSpec((N_REL, H_DIM), lambda i: (0, 0)),
        pl.BlockSpec((H_DIM, H_DIM), lambda i: (0, 0)),
    ],
    out_specs=pl.BlockSpec((BLK, H_DIM), lambda i: (i, 0)),
    out_shape=jax.ShapeDtypeStruct((N_NODES, H_DIM), jnp.float32),
)


def kernel(ent_emb, rel_emb, neigh_w, edge_index, rel_id):
    dst = edge_index[1]
    cnt = _make_hist()(dst, rel_id).reshape(N_NODES, R_PAD)
    return _dense(ent_emb, cnt, rel_emb, neigh_w)
